# baseline (speedup 1.0000x reference)
import jax
import jax.numpy as jnp
from jax.experimental import pallas as pl
from jax.experimental.pallas import tpu as pltpu


_CH = 1024
_ST = 512
_NSLOT = 3


def _interp_pipeline(x_ref, s_hbm, o_ref, buf, sem):
    m, k = s_hbm.shape
    nchunks = m // _CH
    nsub = _CH // _ST

    def chunk_copy(c):
        slot = c % _NSLOT
        return pltpu.make_async_copy(
            s_hbm.at[pl.ds(c * _CH, _CH), :],
            buf.at[slot],
            sem.at[slot],
        )

    def sub_wait(c, j):
        slot = c % _NSLOT
        pltpu.make_async_copy(
            s_hbm.at[pl.ds(c * _CH + j * _ST, _ST), :],
            buf.at[slot].at[pl.ds(j * _ST, _ST), :],
            sem.at[slot],
        ).wait()

    for c in range(_NSLOT):
        chunk_copy(c).start()

    for c in range(nchunks):
        slot = c % _NSLOT
        for j in range(nsub):
            sub_wait(c, j)
            row = c * _CH + j * _ST
            o_ref[pl.ds(row, _ST), :] = jnp.dot(
                buf[slot, j * _ST:(j + 1) * _ST, :], x_ref[...],
                preferred_element_type=jnp.float32)
        if c + _NSLOT < nchunks:
            chunk_copy(c + _NSLOT).start()


def kernel(x_coarse, interp_matrix):
    m, k = interp_matrix.shape
    n = x_coarse.shape[1]
    return pl.pallas_call(
        _interp_pipeline,
        in_specs=[
            pl.BlockSpec(memory_space=pltpu.MemorySpace.VMEM),
            pl.BlockSpec(memory_space=pl.ANY),
        ],
        out_specs=pl.BlockSpec(memory_space=pltpu.MemorySpace.VMEM),
        out_shape=jax.ShapeDtypeStruct((m, n), jnp.float32),
        scratch_shapes=[
            pltpu.VMEM((_NSLOT, _CH, 4096), jnp.float32),
            pltpu.SemaphoreType.DMA((_NSLOT,)),
        ],
    )(x_coarse, interp_matrix)
